# baseline (speedup 1.0000x reference)
import jax
import jax.numpy as jnp
from jax import lax
from jax.experimental import pallas as pl
from jax.experimental.pallas import tpu as pltpu
from jax.experimental.pallas import tpu_sc as plsc

N_NODES = 10000
D = 128
L = 3
BN_EPS = 1e-5

NP = 10240
NC = 2
NS = 16
DH = D // NC
CHUNK = 128
CHUNKS_PT = 160
PASS = 20
EP = NS * CHUNKS_PT * CHUNK
NA = 10112
ROWS_PT = NA // NS
XROWS_PT = NP // NS
_PIECES = [(k * CHUNK, CHUNK) for k in range(ROWS_PT // CHUNK)]
if ROWS_PT % CHUNK:
    _PIECES.append((ROWS_PT - ROWS_PT % CHUNK, ROWS_PT % CHUNK))


def _sc_agg_body(x_hbm, src_hbm, dst_hbm, out_hbm, sidx0, sidx1, didx0,
                 didx1, rows0, rows1, rows2, rows3, xsp, agg, sg0, sg1, sg2,
                 sg3, si0, si1, di0, di1):
    sidxs = [sidx0, sidx1]
    didxs = [didx0, didx1]
    rows = [rows0, rows1, rows2, rows3]
    semg = [sg0, sg1, sg2, sg3]
    semi = [si0, si1]
    semd = [di0, di1]
    cid = lax.axis_index("c")
    sid = lax.axis_index("s")

    with jax.named_scope("stage_x"):
        npiece = XROWS_PT // CHUNK

        def xpiece(k, b):
            off = sid * XROWS_PT + k * CHUNK
            return pltpu.make_async_copy(
                x_hbm.at[pl.ds(off, CHUNK), pl.ds(cid * DH, DH)], rows[b],
                semg[b])

        for k in range(min(4, npiece)):
            xpiece(k, k).start()
        for k in range(npiece):
            xpiece(k, k % 4).wait()
            off = sid * XROWS_PT + k * CHUNK
            pltpu.sync_copy(rows[k % 4], xsp.at[pl.ds(off, CHUNK)])
            if k + 4 < npiece:
                xpiece(k + 4, k % 4).start()

    zero16 = jnp.zeros((16,), jnp.float32)

    def zrow(i, carry):
        for k in range(DH // 16):
            rows0[i, pl.ds(k * 16, 16)] = zero16
        return carry

    with jax.named_scope("zero_agg"):
        lax.fori_loop(0, CHUNK, zrow, 0)
        for off, n in _PIECES:
            pltpu.sync_copy(rows0.at[pl.ds(0, n)],
                            agg.at[pl.ds(sid * ROWS_PT + off, n)])
    plsc.subcore_barrier()

    NBUF = 4

    NPASS = CHUNKS_PT // PASS

    def idx_load(p, q):
        sc = pltpu.make_async_copy(
            src_hbm.at[sid, pl.ds(p * PASS, PASS)], sidxs[q], semi[q])
        dc = pltpu.make_async_copy(
            dst_hbm.at[sid, pl.ds(p * PASS, PASS)], didxs[q], semd[q])
        return sc, dc

    for c in idx_load(0, 0):
        c.start()
    for p in range(NPASS):
        q = p % 2
        sidx, didx = sidxs[q], didxs[q]
        for c in idx_load(p, q):
            c.wait()
        if p + 1 < NPASS:
            for c in idx_load(p + 1, 1 - q):
                c.start()

        def gather(j, b, sidx=sidx):
            return pltpu.make_async_copy(xsp.at[sidx.at[j]], rows[b], semg[b])

        for b in range(NBUF):
            gather(b, b).start()

        def ring_step(t, carry, gather=gather, didx=didx):
            j0 = NBUF * t
            for b in range(NBUF):
                gather(j0 + b, b).wait()
                pltpu.sync_copy(rows[b], agg.at[didx.at[j0 + b]], add=True)

                @pl.when(j0 + b + NBUF < PASS)
                def _(b=b, j0=j0):
                    gather(j0 + b + NBUF, b).start()
            return carry

        lax.fori_loop(0, PASS // NBUF, ring_step, 0)

    plsc.subcore_barrier()

    with jax.named_scope("writeback"):
        for off, n in _PIECES:
            pltpu.sync_copy(agg.at[pl.ds(sid * ROWS_PT + off, n)],
                            rows0.at[pl.ds(0, n)])
            pltpu.sync_copy(
                rows0.at[pl.ds(0, n)],
                out_hbm.at[pl.ds(sid * ROWS_PT + off, n),
                           pl.ds(cid * DH, DH)])


_sc_agg = pl.kernel(
    _sc_agg_body,
    out_type=jax.ShapeDtypeStruct((NA, D), jnp.float32),
    mesh=plsc.VectorSubcoreMesh(core_axis_name="c", subcore_axis_name="s"),
    scratch_types=[
        pltpu.VMEM((PASS, CHUNK), jnp.int32),
        pltpu.VMEM((PASS, CHUNK), jnp.int32),
        pltpu.VMEM((PASS, CHUNK), jnp.int32),
        pltpu.VMEM((PASS, CHUNK), jnp.int32),
        pltpu.VMEM((CHUNK, DH), jnp.float32),
        pltpu.VMEM((CHUNK, DH), jnp.float32),
        pltpu.VMEM((CHUNK, DH), jnp.float32),
        pltpu.VMEM((CHUNK, DH), jnp.float32),
        pltpu.VMEM_SHARED((NP, DH), jnp.float32),
        pltpu.VMEM_SHARED((NA, DH), jnp.float32),
        pltpu.SemaphoreType.DMA,
        pltpu.SemaphoreType.DMA,
        pltpu.SemaphoreType.DMA,
        pltpu.SemaphoreType.DMA,
        pltpu.SemaphoreType.DMA,
        pltpu.SemaphoreType.DMA,
        pltpu.SemaphoreType.DMA,
        pltpu.SemaphoreType.DMA,
    ],
    compiler_params=pltpu.CompilerParams(use_tc_tiling_on_sc=False),
)


def _tc_layer_body(x_ref, p_ref, w1_ref, b1_ref, w2_ref, b2_ref, g_ref,
                   bt_ref, hs_ref, xn_ref):
    z = x_ref[:N_NODES, :] + p_ref[:N_NODES, :]
    h1 = jnp.dot(z, w1_ref[:, :], preferred_element_type=jnp.float32)
    h1 = jnp.maximum(h1 + b1_ref[:, :], 0.0)
    z2 = jnp.dot(h1, w2_ref[:, :], preferred_element_type=jnp.float32)
    z2 = z2 + b2_ref[:, :]
    m = jnp.mean(z2, axis=0, keepdims=True)
    c = z2 - m
    v = jnp.mean(c * c, axis=0, keepdims=True)
    y = jnp.maximum(c * lax.rsqrt(v + BN_EPS) * g_ref[:, :] + bt_ref[:, :], 0.0)
    hs_ref[:, :] = y
    xn_ref[:N_NODES, :] = y
    xn_ref[N_NODES:, :] = jnp.zeros((NP - N_NODES, D), jnp.float32)


_tc_layer = pl.pallas_call(
    _tc_layer_body,
    out_shape=(
        jax.ShapeDtypeStruct((N_NODES, D), jnp.float32),
        jax.ShapeDtypeStruct((NP, D), jnp.float32),
    ),
)


def kernel(h, edge_index, W1, b1, W2, b2, gamma, beta):
    src = edge_index[0]
    dst = edge_index[1]
    pad_e = EP - src.shape[0]
    pad_idx = jnp.full((pad_e,), N_NODES, jnp.int32)
    src_p = jnp.concatenate([src, pad_idx]).reshape(NS, CHUNKS_PT, CHUNK)
    dst_p = jnp.concatenate([dst, pad_idx]).reshape(NS, CHUNKS_PT, CHUNK)
    x = jnp.pad(h, ((0, NP - N_NODES), (0, 0)))
    hs = []
    for i in range(L):
        parts = _sc_agg(x, src_p, dst_p)
        y, x = _tc_layer(x, parts, W1[i], b1[i].reshape(1, D), W2[i],
                         b2[i].reshape(1, D), gamma[i].reshape(1, D),
                         beta[i].reshape(1, D))
        hs.append(y)
    return jnp.stack(hs)
